# Initial kernel scaffold; baseline (speedup 1.0000x reference)
import jax
import jax.numpy as jnp
from jax.experimental import pallas as pl


def kernel(hidden_states, gate_w, W1, b1, W2, b2):
    raise NotImplementedError("write your pallas kernel here")


# speedup vs baseline: 3.5554x; 3.5554x over previous
import functools

import jax
import jax.numpy as jnp
from jax import lax
from jax.experimental import pallas as pl
from jax.experimental.pallas import tpu as pltpu
from jax.experimental.pallas import tpu_sc as plsc

K = 2
BM = 256
NC = 2
NS = 16
NW = NC * NS
CH = 32
CH2 = 16


def _gate_body(x_ref, gw_ref, idx_ref, w_ref):
    x = x_ref[...]
    logits = jnp.dot(x, gw_ref[...], preferred_element_type=jnp.float32)
    n, e = logits.shape
    eidx = lax.broadcasted_iota(jnp.int32, (n, e), 1)
    m1 = jnp.max(logits, axis=1, keepdims=True)
    i1 = jnp.min(jnp.where(logits == m1, eidx, e), axis=1, keepdims=True)
    masked = jnp.where(eidx == i1, -jnp.inf, logits)
    m2 = jnp.max(masked, axis=1, keepdims=True)
    i2 = jnp.min(jnp.where(masked == m2, eidx, e), axis=1, keepdims=True)
    z = jnp.exp(m2 - m1)
    w1 = 1.0 / (1.0 + z)
    idx_ref[...] = jnp.concatenate([i1, i2], axis=1)
    w_ref[...] = jnp.concatenate([w1, 1.0 - w1], axis=1)


_INV_SQRT2 = 0.7071067811865476


def _ffn1_body(be_ref, nv_ref, xg_ref, w1_ref, b1_ref, h_ref, w1b_ref):
    i = pl.program_id(0)
    e = be_ref[i]
    pe = be_ref[jnp.maximum(i - 1, 0)]

    @pl.when((i == 0) | (e != pe))
    def _cast():
        w1b_ref[...] = w1_ref[0].astype(jnp.bfloat16)

    @pl.when(nv_ref[i] > 0)
    def _compute():
        xb = xg_ref[...].astype(jnp.bfloat16)
        h = jnp.dot(xb, w1b_ref[...], preferred_element_type=jnp.float32)
        h = h + b1_ref[0]
        h = 0.5 * h * (1.0 + lax.erf(h * _INV_SQRT2))
        h_ref[...] = h.astype(jnp.bfloat16)


def _ffn2_body(be_ref, nv_ref, h_ref, w2_ref, b2_ref, wr_ref, y_ref, w2b_ref):
    i = pl.program_id(0)
    e = be_ref[i]
    pe = be_ref[jnp.maximum(i - 1, 0)]

    @pl.when((i == 0) | (e != pe))
    def _cast():
        w2b_ref[...] = w2_ref[0].astype(jnp.bfloat16)

    @pl.when(nv_ref[i] > 0)
    def _compute():
        acc = jnp.dot(h_ref[...], w2b_ref[...],
                      preferred_element_type=jnp.float32)
        y_ref[...] = (acc + b2_ref[0]) * wr_ref[...]


def _make_dispatch(n, d, r):
    tokw = n // NW
    mesh = plsc.VectorSubcoreMesh(core_axis_name="c", subcore_axis_name="s")

    @functools.partial(
        pl.kernel, mesh=mesh,
        out_type=jax.ShapeDtypeStruct((r, d), jnp.float32),
        scratch_types=[
            pltpu.VMEM((CH, d), jnp.float32),
            pltpu.VMEM((CH,), jnp.int32),
            pltpu.VMEM((CH,), jnp.int32),
            pltpu.SemaphoreType.DMA,
        ],
    )
    def dispatch(x_hbm, r0_hbm, r1_hbm, xg_hbm, xbuf, i0, i1, sem):
        w = lax.axis_index("s") * NC + lax.axis_index("c")
        base = w * tokw
        for c in range(tokw // CH):
            off = base + c * CH
            pltpu.sync_copy(x_hbm.at[pl.ds(off, CH)], xbuf)
            pltpu.sync_copy(r0_hbm.at[pl.ds(off, CH)], i0)
            pltpu.sync_copy(r1_hbm.at[pl.ds(off, CH)], i1)
            pltpu.async_copy(xbuf, xg_hbm.at[i0], sem).wait()
            pltpu.async_copy(xbuf, xg_hbm.at[i1], sem).wait()

    return dispatch


def _make_combine(n, d):
    tokw = n // NW
    nsl = d // 16
    mesh = plsc.VectorSubcoreMesh(core_axis_name="c", subcore_axis_name="s")

    @functools.partial(
        pl.kernel, mesh=mesh,
        out_type=jax.ShapeDtypeStruct((n, d), jnp.float32),
        scratch_types=[
            pltpu.VMEM((CH2, d), jnp.float32),
            pltpu.VMEM((CH2, d), jnp.float32),
            pltpu.VMEM((CH2,), jnp.int32),
            pltpu.VMEM((CH2,), jnp.int32),
            pltpu.SemaphoreType.DMA,
            pltpu.SemaphoreType.DMA,
        ],
    )
    def combine(y_hbm, r0_hbm, r1_hbm, out_hbm, yb0, yb1, i0, i1, s0, s1):
        w = lax.axis_index("s") * NC + lax.axis_index("c")
        base = w * tokw
        for c in range(tokw // CH2):
            off = base + c * CH2
            pltpu.sync_copy(r0_hbm.at[pl.ds(off, CH2)], i0)
            pltpu.sync_copy(r1_hbm.at[pl.ds(off, CH2)], i1)
            cp0 = pltpu.async_copy(y_hbm.at[i0], yb0, s0)
            cp1 = pltpu.async_copy(y_hbm.at[i1], yb1, s1)
            cp0.wait()
            cp1.wait()

            def add_body(k, _):
                t = k // nsl
                cc = (k % nsl) * 16
                yb0[t, pl.ds(cc, 16)] = (yb0[t, pl.ds(cc, 16)]
                                         + yb1[t, pl.ds(cc, 16)])
                return 0

            lax.fori_loop(0, CH2 * nsl, add_body, 0, unroll=8)
            pltpu.sync_copy(yb0, out_hbm.at[pl.ds(off, CH2)])

    return combine


def kernel(hidden_states, gate_w, W1, b1, W2, b2):
    B, S, D = hidden_states.shape
    E = gate_w.shape[1]
    FF = W1.shape[2]
    N = B * S
    NB = (N * K) // BM + E
    R = NB * BM

    x2d = hidden_states.reshape(N, D)

    idx, tw = pl.pallas_call(
        _gate_body,
        out_shape=[jax.ShapeDtypeStruct((N, K), jnp.int32),
                   jax.ShapeDtypeStruct((N, K), jnp.float32)],
    )(x2d, gate_w)

    ef = idx.reshape(-1)
    wf = tw.reshape(-1)
    onehot = (ef[:, None] == jnp.arange(E, dtype=jnp.int32)[None, :])
    csum = jnp.cumsum(onehot.astype(jnp.int32), axis=0)
    rank = jnp.take_along_axis(csum, ef[:, None], axis=1)[:, 0] - 1
    counts = csum[-1]
    padded = ((counts + BM - 1) // BM) * BM
    pend = jnp.cumsum(padded)
    starts = pend - padded
    rows = starts[ef] + rank
    rows_nk = rows.reshape(N, K)
    row0 = rows_nk[:, 0]
    row1 = rows_nk[:, 1]
    block_starts = jnp.arange(NB, dtype=jnp.int32) * BM
    be = jnp.minimum(jnp.searchsorted(pend, block_starts, side="right"),
                     E - 1).astype(jnp.int32)
    nvalid = jnp.clip(starts[be] + counts[be] - block_starts, 0, BM)
    nvalid = nvalid.astype(jnp.int32)
    wrow = jnp.zeros((R, 1), jnp.float32).at[rows, 0].set(wf)

    xg = _make_dispatch(N, D, R)(x2d, row0, row1)

    grid_spec1 = pltpu.PrefetchScalarGridSpec(
        num_scalar_prefetch=2,
        grid=(NB,),
        in_specs=[
            pl.BlockSpec((BM, D), lambda i, be, nv: (i, 0)),
            pl.BlockSpec((1, D, FF), lambda i, be, nv: (be[i], 0, 0)),
            pl.BlockSpec((1, 1, FF), lambda i, be, nv: (be[i], 0, 0)),
        ],
        out_specs=pl.BlockSpec((BM, FF), lambda i, be, nv: (i, 0)),
        scratch_shapes=[pltpu.VMEM((D, FF), jnp.bfloat16)],
    )
    h = pl.pallas_call(
        _ffn1_body,
        grid_spec=grid_spec1,
        out_shape=jax.ShapeDtypeStruct((R, FF), jnp.bfloat16),
    )(be, nvalid, xg, W1, b1.reshape(E, 1, FF))

    grid_spec2 = pltpu.PrefetchScalarGridSpec(
        num_scalar_prefetch=2,
        grid=(NB,),
        in_specs=[
            pl.BlockSpec((BM, FF), lambda i, be, nv: (i, 0)),
            pl.BlockSpec((1, FF, D), lambda i, be, nv: (be[i], 0, 0)),
            pl.BlockSpec((1, 1, D), lambda i, be, nv: (be[i], 0, 0)),
            pl.BlockSpec((BM, 1), lambda i, be, nv: (i, 0)),
        ],
        out_specs=pl.BlockSpec((BM, D), lambda i, be, nv: (i, 0)),
        scratch_shapes=[pltpu.VMEM((FF, D), jnp.bfloat16)],
    )
    y = pl.pallas_call(
        _ffn2_body,
        grid_spec=grid_spec2,
        out_shape=jax.ShapeDtypeStruct((R, D), jnp.float32),
    )(be, nvalid, h, W2, b2.reshape(E, 1, D), wrow)

    out = _make_combine(N, D)(y, row0, row1)
    return out.reshape(B, S, D)
